# baseline (speedup 1.0000x reference)
import jax
import jax.numpy as jnp
from jax import lax
from jax.experimental import pallas as pl
from jax.experimental.pallas import tpu as pltpu

_H = 224
_W = 224
_HW = _H * _W
_N = 16384
_Q = 256
_NQB = _HW // _Q
_P = 2048
_NP = _N // _P
_NG = 1024
_C = 40
_EPS = 1e-10
_BIG = 3.0e38


def _bf(x):
    return x.astype(jnp.bfloat16).astype(jnp.float32)


def _body(ib_ref, pts_ref, attrs_ref, depth_ref, colors_ref, feats_ref,
          mask_ref, d2n_s, px_s, py_s, psq_s, g1_s, g2_s, g3_s, acc_ref):
    qb = pl.program_id(0)

    xw = _bf(pts_ref[0:1, :])
    yw = _bf(pts_ref[1:2, :])
    zw = _bf(pts_ref[2:3, :])
    u = xw * ib_ref[0, 0] + yw * ib_ref[0, 1] + zw * ib_ref[0, 2]
    v = xw * ib_ref[1, 0] + yw * ib_ref[1, 1] + zw * ib_ref[1, 2]
    den = xw * ib_ref[2, 0] + yw * ib_ref[2, 1] + zw * ib_ref[2, 2]
    den = jnp.maximum(den, 1e-8)
    px = u / den
    py = v / den
    px_s[...] = px
    py_s[...] = py
    psq_s[...] = px * px + py * py

    ids = qb * _Q + lax.broadcasted_iota(jnp.int32, (_Q, 1), 0)
    iy = ids // _W
    ix = ids - iy * _W
    qx = ix.astype(jnp.float32) + 0.5
    qy = iy.astype(jnp.float32) + 0.5
    bqx = _bf(qx)
    bqy = _bf(qy)
    qq = qx * qx + qy * qy

    for t in range(_NP):
        sl = slice(t * _P, (t + 1) * _P)
        pxt = px_s[:, sl]
        pyt = py_s[:, sl]
        m = bqx * _bf(pxt) + bqy * _bf(pyt)
        d2 = (qq - 2.0 * m) + psq_s[:, sl]
        d2n_s[:, sl] = d2
        m1 = d2[:, 0:128]
        for g in range(1, 16):
            m1 = jnp.minimum(m1, d2[:, g * 128:(g + 1) * 128])
        m2 = jnp.full_like(m1, _BIG)
        for g in range(16):
            x = d2[:, g * 128:(g + 1) * 128]
            m2 = jnp.minimum(m2, jnp.where(x == m1, _BIG, x))
        m3 = jnp.full_like(m1, _BIG)
        for g in range(16):
            x = d2[:, g * 128:(g + 1) * 128]
            m3 = jnp.minimum(m3, jnp.where((x == m1) | (x == m2), _BIG, x))
        gsl = slice(t * 128, (t + 1) * 128)
        g1_s[:, gsl] = m1
        g2_s[:, gsl] = m2
        g3_s[:, gsl] = m3

    def _round(_, carry):
        big_t, cnt = carry
        g1 = g1_s[...]
        active = cnt < 16
        mn = jnp.min(g1, axis=1, keepdims=True)
        hit = (g1 == mn) & active
        nh = jnp.sum(hit.astype(jnp.int32), axis=1, keepdims=True)
        big_t = jnp.where(active, mn, big_t)
        cnt = cnt + nh
        g2 = g2_s[...]
        g3 = g3_s[...]
        g1_s[...] = jnp.where(hit, g2, g1)
        g2_s[...] = jnp.where(hit, g3, g2)
        g3_s[...] = jnp.where(hit, _BIG, g3)
        return big_t, cnt

    thr0 = jnp.full((_Q, 1), _BIG, jnp.float32)
    cnt0 = jnp.zeros((_Q, 1), jnp.int32)
    thr, _ = lax.fori_loop(0, 16, _round, (thr0, cnt0))

    acc_ref[...] = jnp.zeros_like(acc_ref)
    for t in range(_NP):
        sl = slice(t * _P, (t + 1) * _P)
        x = d2n_s[:, sl]
        dx = qx - px_s[:, sl]
        dy = qy - py_s[:, sl]
        s = dx * dx + dy * dy + 1e-12
        w = jnp.where((x <= thr) & (s < 4.0), jnp.exp(-s), 0.0)
        acc_ref[...] += jnp.dot(w, attrs_ref[sl, :],
                                preferred_element_type=jnp.float32,
                                precision=lax.Precision.HIGHEST)

    a = acc_ref[...]
    wsum = a[:, 0:1]
    denom = wsum + _EPS
    depth_ref[...] = a[:, 1:2] / denom
    colors_ref[...] = a[:, 2:5] / denom
    feats_ref[...] = a[:, 5:37] / denom
    mask_ref[...] = wsum > 0.0


def kernel(pcd_points, pcd_colors, pcd_feats, intrinsics):
    pts_t = pcd_points.T
    ib = intrinsics.astype(jnp.bfloat16).astype(jnp.float32)
    ones = jnp.ones((_N, 1), jnp.float32)
    depth = pcd_points[:, 2:3]
    pad = jnp.zeros((_N, 3), jnp.float32)
    attrs = jnp.concatenate([ones, depth, pcd_colors, pcd_feats, pad], axis=1)

    out = pl.pallas_call(
        _body,
        grid=(_NQB,),
        in_specs=[
            pl.BlockSpec(memory_space=pltpu.SMEM),
            pl.BlockSpec((3, _N), lambda qb: (0, 0)),
            pl.BlockSpec((_N, _C), lambda qb: (0, 0)),
        ],
        out_specs=[
            pl.BlockSpec((_Q, 1), lambda qb: (qb, 0)),
            pl.BlockSpec((_Q, 3), lambda qb: (qb, 0)),
            pl.BlockSpec((_Q, 32), lambda qb: (qb, 0)),
            pl.BlockSpec((_Q, 1), lambda qb: (qb, 0)),
        ],
        out_shape=[
            jax.ShapeDtypeStruct((_HW, 1), jnp.float32),
            jax.ShapeDtypeStruct((_HW, 3), jnp.float32),
            jax.ShapeDtypeStruct((_HW, 32), jnp.float32),
            jax.ShapeDtypeStruct((_HW, 1), jnp.bool_),
        ],
        scratch_shapes=[
            pltpu.VMEM((_Q, _N), jnp.float32),
            pltpu.VMEM((1, _N), jnp.float32),
            pltpu.VMEM((1, _N), jnp.float32),
            pltpu.VMEM((1, _N), jnp.float32),
            pltpu.VMEM((_Q, _NG), jnp.float32),
            pltpu.VMEM((_Q, _NG), jnp.float32),
            pltpu.VMEM((_Q, _NG), jnp.float32),
            pltpu.VMEM((_Q, _C), jnp.float32),
        ],
        compiler_params=pltpu.CompilerParams(
            dimension_semantics=("arbitrary",),
        ),
    )(ib, pts_t, attrs)

    depths, colors, feats, masks = out
    return (depths.reshape(_H, _W), colors.reshape(_H, _W, 3),
            feats.reshape(_H, _W, 32), masks.reshape(_H, _W))


# speedup vs baseline: 1.4897x; 1.4897x over previous
import jax
import jax.numpy as jnp
from jax import lax
from jax.experimental import pallas as pl
from jax.experimental.pallas import tpu as pltpu

_H = 224
_W = 224
_HW = _H * _W
_N = 16384
_Q = 256
_NQB = _HW // _Q
_P = 2048
_NP = _N // _P
_NG = 1024
_C = 40
_EPS = 1e-10
_BIG = 3.0e38


def _bf(x):
    return x.astype(jnp.bfloat16).astype(jnp.float32)


def _body(ib_ref, pts_ref, attrs_ref, depth_ref, colors_ref, feats_ref,
          mask_ref, d2n_s, px_s, py_s, psq_s, g1_s, g2_s, g3_s,
          h1_s, h2_s, h3_s, h4_s, h5_s, h6_s, acc_ref):
    hs_list = [h1_s, h2_s, h3_s, h4_s, h5_s, h6_s]
    qb = pl.program_id(0)

    xw = _bf(pts_ref[0:1, :])
    yw = _bf(pts_ref[1:2, :])
    zw = _bf(pts_ref[2:3, :])
    u = xw * ib_ref[0, 0] + yw * ib_ref[0, 1] + zw * ib_ref[0, 2]
    v = xw * ib_ref[1, 0] + yw * ib_ref[1, 1] + zw * ib_ref[1, 2]
    den = xw * ib_ref[2, 0] + yw * ib_ref[2, 1] + zw * ib_ref[2, 2]
    den = jnp.maximum(den, 1e-8)
    px = u / den
    py = v / den
    px_s[...] = px
    py_s[...] = py
    psq_s[...] = px * px + py * py

    ids = qb * _Q + lax.broadcasted_iota(jnp.int32, (_Q, 1), 0)
    iy = ids // _W
    ix = ids - iy * _W
    qx = ix.astype(jnp.float32) + 0.5
    qy = iy.astype(jnp.float32) + 0.5
    bqx = _bf(qx)
    bqy = _bf(qy)
    qq = qx * qx + qy * qy

    for t in range(_NP):
        sl = slice(t * _P, (t + 1) * _P)
        pxt = px_s[:, sl]
        pyt = py_s[:, sl]
        m = bqx * _bf(pxt) + bqy * _bf(pyt)
        d2 = (qq - 2.0 * m) + psq_s[:, sl]
        d2n_s[:, sl] = d2
        m1 = d2[:, 0:128]
        for g in range(1, 16):
            m1 = jnp.minimum(m1, d2[:, g * 128:(g + 1) * 128])
        m2 = jnp.full_like(m1, _BIG)
        for g in range(16):
            x = d2[:, g * 128:(g + 1) * 128]
            m2 = jnp.minimum(m2, jnp.where(x == m1, _BIG, x))
        m3 = jnp.full_like(m1, _BIG)
        for g in range(16):
            x = d2[:, g * 128:(g + 1) * 128]
            m3 = jnp.minimum(m3, jnp.where((x == m1) | (x == m2), _BIG, x))
        gsl = slice(t * 128, (t + 1) * 128)
        g1_s[:, gsl] = m1
        g2_s[:, gsl] = m2
        g3_s[:, gsl] = m3

    h = [g1_s[:, 0:128]] + [jnp.full((_Q, 128), _BIG, jnp.float32)] * 5
    for src in (g1_s, g2_s, g3_s):
        for t in range(_NP):
            if src is g1_s and t == 0:
                continue
            x = src[:, t * 128:(t + 1) * 128]
            c = [x < m for m in h]
            nh_ = [None] * 6
            for k in range(5, 0, -1):
                nh_[k] = jnp.where(c[k - 1], h[k - 1],
                                   jnp.where(c[k], x, h[k]))
            nh_[0] = jnp.where(c[0], x, h[0])
            h = nh_
    for k in range(6):
        hs_list[k][...] = h[k]

    def _round(_, carry):
        big_t, cnt = carry
        v = [r[...] for r in hs_list]
        active = cnt < 16
        mn = jnp.min(v[0], axis=1, keepdims=True)
        hit = (v[0] == mn) & active
        nh = jnp.sum(hit.astype(jnp.int32), axis=1, keepdims=True)
        big_t = jnp.where(active, mn, big_t)
        cnt = cnt + nh
        for k in range(5):
            hs_list[k][...] = jnp.where(hit, v[k + 1], v[k])
        hs_list[5][...] = jnp.where(hit, _BIG, v[5])
        return big_t, cnt

    thr0 = jnp.full((_Q, 1), _BIG, jnp.float32)
    cnt0 = jnp.zeros((_Q, 1), jnp.int32)
    thr, _ = lax.fori_loop(0, 16, _round, (thr0, cnt0))

    acc_ref[...] = jnp.zeros_like(acc_ref)
    for t in range(_NP):
        sl = slice(t * _P, (t + 1) * _P)
        x = d2n_s[:, sl]
        dx = qx - px_s[:, sl]
        dy = qy - py_s[:, sl]
        s = dx * dx + dy * dy + 1e-12
        w = jnp.where((x <= thr) & (s < 4.0), jnp.exp(-s), 0.0)
        acc_ref[...] += jnp.dot(w.astype(jnp.bfloat16), attrs_ref[sl, :],
                                preferred_element_type=jnp.float32)

    a = acc_ref[...]
    wsum = a[:, 0:1]
    denom = wsum + _EPS
    depth_ref[...] = a[:, 1:2] / denom
    colors_ref[...] = a[:, 2:5] / denom
    feats_ref[...] = a[:, 5:37] / denom
    mask_ref[...] = wsum > 0.0


def kernel(pcd_points, pcd_colors, pcd_feats, intrinsics):
    pts_t = pcd_points.T
    ib = intrinsics.astype(jnp.bfloat16).astype(jnp.float32)
    ones = jnp.ones((_N, 1), jnp.float32)
    depth = pcd_points[:, 2:3]
    pad = jnp.zeros((_N, 3), jnp.float32)
    attrs = jnp.concatenate([ones, depth, pcd_colors, pcd_feats, pad],
                            axis=1).astype(jnp.bfloat16)

    out = pl.pallas_call(
        _body,
        grid=(_NQB,),
        in_specs=[
            pl.BlockSpec(memory_space=pltpu.SMEM),
            pl.BlockSpec((3, _N), lambda qb: (0, 0)),
            pl.BlockSpec((_N, _C), lambda qb: (0, 0)),
        ],
        out_specs=[
            pl.BlockSpec((_Q, 1), lambda qb: (qb, 0)),
            pl.BlockSpec((_Q, 3), lambda qb: (qb, 0)),
            pl.BlockSpec((_Q, 32), lambda qb: (qb, 0)),
            pl.BlockSpec((_Q, 1), lambda qb: (qb, 0)),
        ],
        out_shape=[
            jax.ShapeDtypeStruct((_HW, 1), jnp.float32),
            jax.ShapeDtypeStruct((_HW, 3), jnp.float32),
            jax.ShapeDtypeStruct((_HW, 32), jnp.float32),
            jax.ShapeDtypeStruct((_HW, 1), jnp.bool_),
        ],
        scratch_shapes=[
            pltpu.VMEM((_Q, _N), jnp.float32),
            pltpu.VMEM((1, _N), jnp.float32),
            pltpu.VMEM((1, _N), jnp.float32),
            pltpu.VMEM((1, _N), jnp.float32),
            pltpu.VMEM((_Q, _NG), jnp.float32),
            pltpu.VMEM((_Q, _NG), jnp.float32),
            pltpu.VMEM((_Q, _NG), jnp.float32),
            pltpu.VMEM((_Q, 128), jnp.float32),
            pltpu.VMEM((_Q, 128), jnp.float32),
            pltpu.VMEM((_Q, 128), jnp.float32),
            pltpu.VMEM((_Q, 128), jnp.float32),
            pltpu.VMEM((_Q, 128), jnp.float32),
            pltpu.VMEM((_Q, 128), jnp.float32),
            pltpu.VMEM((_Q, _C), jnp.float32),
        ],
        compiler_params=pltpu.CompilerParams(
            dimension_semantics=("arbitrary",),
        ),
    )(ib, pts_t, attrs)

    depths, colors, feats, masks = out
    return (depths.reshape(_H, _W), colors.reshape(_H, _W, 3),
            feats.reshape(_H, _W, 32), masks.reshape(_H, _W))


# speedup vs baseline: 1.6363x; 1.0985x over previous
import jax
import jax.numpy as jnp
from jax import lax
from jax.experimental import pallas as pl
from jax.experimental.pallas import tpu as pltpu

_H = 224
_W = 224
_HW = _H * _W
_N = 16384
_Q = 256
_NQB = _HW // _Q
_P = 2048
_NP = _N // _P
_NG = 1024
_C = 40
_EPS = 1e-10
_BIG = 3.0e38


def _bf(x):
    return x.astype(jnp.bfloat16).astype(jnp.float32)


def _body(ib_ref, pts_ref, attrs_ref, depth_ref, colors_ref, feats_ref,
          mask_ref, d2n_s, px_s, py_s, psq_s, g1_s, g2_s, g3_s,
          h1_s, h2_s, h3_s, h4_s, h5_s, h6_s, acc_ref):
    hs_list = [h1_s, h2_s, h3_s, h4_s, h5_s, h6_s]
    qb = pl.program_id(0)

    xw = _bf(pts_ref[0:1, :])
    yw = _bf(pts_ref[1:2, :])
    zw = _bf(pts_ref[2:3, :])
    u = xw * ib_ref[0, 0] + yw * ib_ref[0, 1] + zw * ib_ref[0, 2]
    v = xw * ib_ref[1, 0] + yw * ib_ref[1, 1] + zw * ib_ref[1, 2]
    den = xw * ib_ref[2, 0] + yw * ib_ref[2, 1] + zw * ib_ref[2, 2]
    den = jnp.maximum(den, 1e-8)
    px = u / den
    py = v / den
    px_s[...] = px
    py_s[...] = py
    psq_s[...] = px * px + py * py

    ids = qb * _Q + lax.broadcasted_iota(jnp.int32, (_Q, 1), 0)
    iy = ids // _W
    ix = ids - iy * _W
    qx = ix.astype(jnp.float32) + 0.5
    qy = iy.astype(jnp.float32) + 0.5
    bqx = _bf(qx)
    bqy = _bf(qy)
    qq = qx * qx + qy * qy
    qb2 = jnp.concatenate([bqx, bqy], axis=1).astype(jnp.bfloat16)

    for t in range(_NP):
        sl = slice(t * _P, (t + 1) * _P)
        pxt = px_s[:, sl]
        pyt = py_s[:, sl]
        pb2 = jnp.concatenate([_bf(pxt) * -2.0, _bf(pyt) * -2.0],
                              axis=0).astype(jnp.bfloat16)
        m2x = jnp.dot(qb2, pb2, preferred_element_type=jnp.float32)
        d2 = (qq + m2x) + psq_s[:, sl]
        d2n_s[:, sl] = d2
        m1 = d2[:, 0:128]
        for g in range(1, 16):
            m1 = jnp.minimum(m1, d2[:, g * 128:(g + 1) * 128])
        m2 = jnp.full_like(m1, _BIG)
        for g in range(16):
            x = d2[:, g * 128:(g + 1) * 128]
            m2 = jnp.minimum(m2, jnp.where(x == m1, _BIG, x))
        m3 = jnp.full_like(m1, _BIG)
        for g in range(16):
            x = d2[:, g * 128:(g + 1) * 128]
            m3 = jnp.minimum(m3, jnp.where((x == m1) | (x == m2), _BIG, x))
        gsl = slice(t * 128, (t + 1) * 128)
        g1_s[:, gsl] = m1
        g2_s[:, gsl] = m2
        g3_s[:, gsl] = m3

    h = [g1_s[:, 0:128]] + [jnp.full((_Q, 128), _BIG, jnp.float32)] * 5
    for src in (g1_s, g2_s, g3_s):
        for t in range(_NP):
            if src is g1_s and t == 0:
                continue
            x = src[:, t * 128:(t + 1) * 128]
            c = [x < m for m in h]
            nh_ = [None] * 6
            for k in range(5, 0, -1):
                nh_[k] = jnp.where(c[k - 1], h[k - 1],
                                   jnp.where(c[k], x, h[k]))
            nh_[0] = jnp.where(c[0], x, h[0])
            h = nh_
    for k in range(6):
        hs_list[k][...] = h[k]

    def _round(_, carry):
        big_t, cnt = carry
        v = [r[...] for r in hs_list]
        active = cnt < 16
        mn = jnp.min(v[0], axis=1, keepdims=True)
        hit = (v[0] == mn) & active
        nh = jnp.sum(hit.astype(jnp.int32), axis=1, keepdims=True)
        big_t = jnp.where(active, mn, big_t)
        cnt = cnt + nh
        for k in range(5):
            hs_list[k][...] = jnp.where(hit, v[k + 1], v[k])
        hs_list[5][...] = jnp.where(hit, _BIG, v[5])
        return big_t, cnt

    thr0 = jnp.full((_Q, 1), _BIG, jnp.float32)
    cnt0 = jnp.zeros((_Q, 1), jnp.int32)
    thr, _ = lax.fori_loop(0, 16, _round, (thr0, cnt0))

    acc_ref[...] = jnp.zeros_like(acc_ref)
    for t in range(_NP):
        sl = slice(t * _P, (t + 1) * _P)
        x = d2n_s[:, sl]
        dx = qx - px_s[:, sl]
        dy = qy - py_s[:, sl]
        s = dx * dx + dy * dy + 1e-12
        w = jnp.where((x <= thr) & (s < 4.0), jnp.exp(-s), 0.0)
        acc_ref[...] += jnp.dot(w.astype(jnp.bfloat16), attrs_ref[sl, :],
                                preferred_element_type=jnp.float32)

    a = acc_ref[...]
    wsum = a[:, 0:1]
    denom = wsum + _EPS
    depth_ref[...] = a[:, 1:2] / denom
    colors_ref[...] = a[:, 2:5] / denom
    feats_ref[...] = a[:, 5:37] / denom
    mask_ref[...] = wsum > 0.0


def kernel(pcd_points, pcd_colors, pcd_feats, intrinsics):
    pts_t = pcd_points.T
    ib = intrinsics.astype(jnp.bfloat16).astype(jnp.float32)
    ones = jnp.ones((_N, 1), jnp.float32)
    depth = pcd_points[:, 2:3]
    pad = jnp.zeros((_N, 3), jnp.float32)
    attrs = jnp.concatenate([ones, depth, pcd_colors, pcd_feats, pad],
                            axis=1).astype(jnp.bfloat16)

    out = pl.pallas_call(
        _body,
        grid=(_NQB,),
        in_specs=[
            pl.BlockSpec(memory_space=pltpu.SMEM),
            pl.BlockSpec((3, _N), lambda qb: (0, 0)),
            pl.BlockSpec((_N, _C), lambda qb: (0, 0)),
        ],
        out_specs=[
            pl.BlockSpec((_Q, 1), lambda qb: (qb, 0)),
            pl.BlockSpec((_Q, 3), lambda qb: (qb, 0)),
            pl.BlockSpec((_Q, 32), lambda qb: (qb, 0)),
            pl.BlockSpec((_Q, 1), lambda qb: (qb, 0)),
        ],
        out_shape=[
            jax.ShapeDtypeStruct((_HW, 1), jnp.float32),
            jax.ShapeDtypeStruct((_HW, 3), jnp.float32),
            jax.ShapeDtypeStruct((_HW, 32), jnp.float32),
            jax.ShapeDtypeStruct((_HW, 1), jnp.bool_),
        ],
        scratch_shapes=[
            pltpu.VMEM((_Q, _N), jnp.float32),
            pltpu.VMEM((1, _N), jnp.float32),
            pltpu.VMEM((1, _N), jnp.float32),
            pltpu.VMEM((1, _N), jnp.float32),
            pltpu.VMEM((_Q, _NG), jnp.float32),
            pltpu.VMEM((_Q, _NG), jnp.float32),
            pltpu.VMEM((_Q, _NG), jnp.float32),
            pltpu.VMEM((_Q, 128), jnp.float32),
            pltpu.VMEM((_Q, 128), jnp.float32),
            pltpu.VMEM((_Q, 128), jnp.float32),
            pltpu.VMEM((_Q, 128), jnp.float32),
            pltpu.VMEM((_Q, 128), jnp.float32),
            pltpu.VMEM((_Q, 128), jnp.float32),
            pltpu.VMEM((_Q, _C), jnp.float32),
        ],
        compiler_params=pltpu.CompilerParams(
            dimension_semantics=("arbitrary",),
        ),
    )(ib, pts_t, attrs)

    depths, colors, feats, masks = out
    return (depths.reshape(_H, _W), colors.reshape(_H, _W, 3),
            feats.reshape(_H, _W, 32), masks.reshape(_H, _W))
